# baseline (speedup 1.0000x reference)
import functools

import jax
import jax.numpy as jnp
from jax import lax
from jax.experimental import pallas as pl
from jax.experimental.pallas import tpu as pltpu
from jax.experimental.pallas import tpu_sc as plsc

B, S, D = 4, 8192, 4096
NW = 32
POS_PER_W = S // NW
CH = 4
NCHUNK = POS_PER_W // CH
VECS = CH * D // 16
UNROLL = 8
TC_B = 3


def _tc_body(seq_ref, tab_ref, out_ref):
    out_ref[...] = seq_ref[...] + tab_ref[...][None, :, :]


def _tc_add(seq, pos_table):
    CHUNK = 128
    grid = (S // CHUNK,)
    return pl.pallas_call(
        _tc_body,
        grid=grid,
        in_specs=[
            pl.BlockSpec((TC_B, CHUNK, D), lambda i: (0, i, 0)),
            pl.BlockSpec((CHUNK, D), lambda i: (i, 0)),
        ],
        out_specs=pl.BlockSpec((TC_B, CHUNK, D), lambda i: (0, i, 0)),
        out_shape=jax.ShapeDtypeStruct((TC_B, S, D), seq.dtype),
        compiler_params=pltpu.CompilerParams(
            dimension_semantics=("parallel",),
        ),
    )(seq, pos_table)


def _sc_body(seq_hbm, tab_hbm, out_hbm,
             t0, t1, t2, s0b, s1b, s2b,
             mt0, mt1, mt2, ms0, ms1, ms2, mo0, mo1, mo2):
    tb = (t0, t1, t2)
    sb = (s0b, s1b, s2b)
    mt = (mt0, mt1, mt2)
    ms = (ms0, ms1, ms2)
    mo = (mo0, mo1, mo2)

    wid = lax.axis_index("s") * 2 + lax.axis_index("c")
    base = wid * POS_PER_W

    def tslice(c):
        return tab_hbm.at[pl.ds(base + c * CH, CH), :]

    def qslice(c):
        return seq_hbm.at[TC_B, pl.ds(base + c * CH, CH), :]

    def oslice(c):
        return out_hbm.at[0, pl.ds(base + c * CH, CH), :]

    def issue_in(c, j):
        pltpu.async_copy(tslice(c), tb[j], mt[j])
        pltpu.async_copy(qslice(c), sb[j], ms[j])

    issue_in(0, 0)
    issue_in(1, 1)

    def compute(j):
        def add(i, _):
            for k in range(UNROLL):
                r = i * UNROLL + k
                row = r // (D // 16)
                col = (r % (D // 16)) * 16
                sb[j][row, pl.ds(col, 16)] = (
                    sb[j][row, pl.ds(col, 16)] + tb[j][row, pl.ds(col, 16)]
                )
            return 0

        lax.fori_loop(0, VECS // UNROLL, add, 0)

    def body(g, _):
        for j in range(3):
            c = g * 3 + j
            jn = (j + 2) % 3
            nxt = c + 2

            def prefetch():
                pltpu.make_async_copy(sb[jn], oslice(nxt), mo[jn]).wait()
                issue_in(nxt, jn)

            def prefetch_first():
                issue_in(nxt, jn)

            if j == 0:
                lax.cond(g > 0,
                         lambda: pl.when(nxt < NCHUNK)(prefetch),
                         lambda: pl.when(nxt < NCHUNK)(prefetch_first))
            else:
                pl.when(nxt < NCHUNK)(prefetch)

            pltpu.make_async_copy(tslice(c), tb[j], mt[j]).wait()
            pltpu.make_async_copy(qslice(c), sb[j], ms[j]).wait()
            compute(j)
            pltpu.async_copy(sb[j], oslice(c), mo[j])
        return 0

    lax.fori_loop(0, NCHUNK // 3, body, 0)

    for c in range((NCHUNK // 3) * 3, NCHUNK):
        j = c % 3
        pltpu.make_async_copy(tslice(c), tb[j], mt[j]).wait()
        pltpu.make_async_copy(qslice(c), sb[j], ms[j]).wait()
        compute(j)
        pltpu.async_copy(sb[j], oslice(c), mo[j])
    for j in range(3):
        c = NCHUNK - 3 + j
        pltpu.make_async_copy(sb[c % 3], oslice(c), mo[c % 3]).wait()


def _sc_add(seq, tab):
    mesh = plsc.VectorSubcoreMesh(core_axis_name="c", subcore_axis_name="s")
    return functools.partial(
        pl.kernel,
        mesh=mesh,
        out_type=jax.ShapeDtypeStruct((B - TC_B, S, D), jnp.float32),
        scratch_types=(
            [pltpu.VMEM((CH, D), jnp.float32)] * 6
            + [pltpu.SemaphoreType.DMA] * 9
        ),
        compiler_params=pltpu.CompilerParams(use_tc_tiling_on_sc=True),
    )(_sc_body)(seq, tab)


@jax.jit
def _pos_add(seq, tab):
    sc_out = _sc_add(seq, tab)
    tc_out = _tc_add(seq, tab)
    return jnp.concatenate([tc_out, sc_out], axis=0)


def kernel(seq, pos_table):
    s = seq.shape[1]
    return _pos_add(seq, pos_table[:s, :])


# speedup vs baseline: 1.0005x; 1.0005x over previous
import functools

import jax
import jax.numpy as jnp
from jax import lax
from jax.experimental import pallas as pl
from jax.experimental.pallas import tpu as pltpu
from jax.experimental.pallas import tpu_sc as plsc

B, S, D = 4, 8192, 4096
NW = 32
POS_PER_W = S // NW
CH = 4
NCHUNK = POS_PER_W // CH
VECS = CH * D // 16
UNROLL = 8
TC_B = 3


def _tc_body(seq_ref, tab_ref, out_ref):
    out_ref[...] = seq_ref[...] + tab_ref[...][None, :, :]


def _tc_add(seq, pos_table):
    CHUNK = 128
    grid = (S // CHUNK,)
    return pl.pallas_call(
        _tc_body,
        grid=grid,
        in_specs=[
            pl.BlockSpec((TC_B, CHUNK, D), lambda i: (0, i, 0)),
            pl.BlockSpec((CHUNK, D), lambda i: (i, 0)),
        ],
        out_specs=pl.BlockSpec((TC_B, CHUNK, D), lambda i: (0, i, 0)),
        out_shape=jax.ShapeDtypeStruct((TC_B, S, D), seq.dtype),
        compiler_params=pltpu.CompilerParams(
            dimension_semantics=("parallel",),
        ),
    )(seq, pos_table)


def _sc_body(seq_hbm, tab_hbm, out_hbm,
             t0, t1, t2, s0b, s1b, s2b,
             mt0, mt1, mt2, ms0, ms1, ms2, mo0, mo1, mo2):
    tb = (t0, t1, t2)
    sb = (s0b, s1b, s2b)
    mt = (mt0, mt1, mt2)
    ms = (ms0, ms1, ms2)
    mo = (mo0, mo1, mo2)

    wid = lax.axis_index("s") * 2 + lax.axis_index("c")
    base = wid * POS_PER_W

    def tslice(c):
        return tab_hbm.at[pl.ds(base + c * CH, CH), :]

    def qslice(c):
        return seq_hbm.at[TC_B, pl.ds(base + c * CH, CH), :]

    def oslice(c):
        return out_hbm.at[0, pl.ds(base + c * CH, CH), :]

    def issue_in(c, j):
        pltpu.async_copy(tslice(c), tb[j], mt[j])
        pltpu.async_copy(qslice(c), sb[j], ms[j])

    issue_in(0, 0)
    issue_in(1, 1)

    def compute(j):
        def add(i, _):
            for k in range(UNROLL):
                r = i * UNROLL + k
                row = r // (D // 16)
                col = (r % (D // 16)) * 16
                sb[j][row, pl.ds(col, 16)] = (
                    sb[j][row, pl.ds(col, 16)] + tb[j][row, pl.ds(col, 16)]
                )
            return 0

        lax.fori_loop(0, VECS // UNROLL, add, 0)

    def body(g, _):
        for j in range(3):
            c = g * 3 + j
            jn = (j + 2) % 3
            nxt = c + 2

            def prefetch():
                pltpu.make_async_copy(sb[jn], oslice(nxt), mo[jn]).wait()
                issue_in(nxt, jn)

            def prefetch_first():
                issue_in(nxt, jn)

            if j == 0:
                lax.cond(g > 0,
                         lambda: pl.when(nxt < NCHUNK)(prefetch),
                         lambda: pl.when(nxt < NCHUNK)(prefetch_first))
            else:
                pl.when(nxt < NCHUNK)(prefetch)

            pltpu.make_async_copy(tslice(c), tb[j], mt[j]).wait()
            pltpu.make_async_copy(qslice(c), sb[j], ms[j]).wait()
            compute(j)
            pltpu.async_copy(sb[j], oslice(c), mo[j])
        return 0

    lax.fori_loop(0, NCHUNK // 3, body, 0)

    for c in range((NCHUNK // 3) * 3, NCHUNK):
        j = c % 3
        pltpu.make_async_copy(tslice(c), tb[j], mt[j]).wait()
        pltpu.make_async_copy(qslice(c), sb[j], ms[j]).wait()
        compute(j)
        pltpu.async_copy(sb[j], oslice(c), mo[j])
    for j in range(3):
        c = NCHUNK - 3 + j
        pltpu.make_async_copy(sb[c % 3], oslice(c), mo[c % 3]).wait()


def _sc_add(seq, tab):
    mesh = plsc.VectorSubcoreMesh(
        core_axis_name="c", subcore_axis_name="s", num_cores=2
    )
    return functools.partial(
        pl.kernel,
        mesh=mesh,
        out_type=jax.ShapeDtypeStruct((B - TC_B, S, D), jnp.float32),
        scratch_types=(
            [pltpu.VMEM((CH, D), jnp.float32)] * 6
            + [pltpu.SemaphoreType.DMA] * 9
        ),
        compiler_params=pltpu.CompilerParams(use_tc_tiling_on_sc=True),
    )(_sc_body)(seq, tab)


@jax.jit
def _pos_add(seq, tab):
    sc_out = _sc_add(seq, tab)
    tc_out = _tc_add(seq, tab)
    return jnp.concatenate([tc_out, sc_out], axis=0)


def kernel(seq, pos_table):
    s = seq.shape[1]
    return _pos_add(seq, pos_table[:s, :])


# speedup vs baseline: 2.0833x; 2.0822x over previous
import functools

import jax
import jax.numpy as jnp
from jax.experimental import pallas as pl
from jax.experimental.pallas import tpu as pltpu


def _add_body(seq_ref, tab_ref, out_ref):
    out_ref[...] = seq_ref[...] + tab_ref[...][None, :, :]


@functools.partial(jax.jit, static_argnames=("chunk", "vmem"))
def _pos_add(seq, pos_table, chunk=128, vmem=None):
    B, S, D = seq.shape
    grid = (S // chunk,)
    params = dict(dimension_semantics=("parallel",))
    if vmem is not None:
        params["vmem_limit_bytes"] = vmem
    return pl.pallas_call(
        _add_body,
        grid=grid,
        in_specs=[
            pl.BlockSpec((B, chunk, D), lambda i: (0, i, 0)),
            pl.BlockSpec((chunk, D), lambda i: (i, 0)),
        ],
        out_specs=pl.BlockSpec((B, chunk, D), lambda i: (0, i, 0)),
        out_shape=jax.ShapeDtypeStruct((B, S, D), seq.dtype),
        compiler_params=pltpu.CompilerParams(**params),
    )(seq, pos_table)


def kernel(seq, pos_table):
    s = seq.shape[1]
    return _pos_add(seq, pos_table[:s, :], chunk=128)


# speedup vs baseline: 2.0842x; 1.0004x over previous
import jax
from jax.experimental import pallas as pl
from jax.experimental.pallas import tpu as pltpu

CHUNK = 128


def _add_body(seq_ref, tab_ref, out_ref):
    out_ref[...] = seq_ref[...] + tab_ref[...][None, :, :]


@jax.jit
def _pos_add(seq, pos_table):
    B, S, D = seq.shape
    grid = (S // CHUNK,)
    return pl.pallas_call(
        _add_body,
        grid=grid,
        in_specs=[
            pl.BlockSpec((B, CHUNK, D), lambda i: (0, i, 0)),
            pl.BlockSpec((CHUNK, D), lambda i: (i, 0)),
        ],
        out_specs=pl.BlockSpec((B, CHUNK, D), lambda i: (0, i, 0)),
        out_shape=jax.ShapeDtypeStruct((B, S, D), seq.dtype),
        compiler_params=pltpu.CompilerParams(
            dimension_semantics=("parallel",),
        ),
    )(seq, pos_table)


def kernel(seq, pos_table):
    s = seq.shape[1]
    return _pos_add(seq, pos_table[:s, :])
